# baseline (speedup 1.0000x reference)
import functools

import jax
import jax.numpy as jnp
from jax import lax
from jax.experimental import pallas as pl
from jax.experimental.pallas import tpu as pltpu
from jax.experimental.pallas import tpu_sc as plsc

B = 16384
M = 4096
NW = 32
RPW = B // NW
L = 16
CHROWS = 4
NCH = RPW // CHROWS
NBUF = 6
IN_AHEAD = 2
OUT_WAIT = NBUF - IN_AHEAD


def _sc_body(state_hbm, choice_hbm, out_hbm, choice_v, *rest):
    bufs = list(rest[:NBUF])
    sin = list(rest[NBUF:2 * NBUF])
    sout = list(rest[2 * NBUF:3 * NBUF])
    wid = lax.axis_index("s") * 2 + lax.axis_index("c")
    base = wid * RPW

    pltpu.sync_copy(choice_hbm.at[pl.ds(base, RPW)], choice_v.at[pl.ds(0, RPW)])

    lane = lax.iota(jnp.int32, L)
    row_idx = lane & (CHROWS - 1)
    mask = lane < CHROWS
    ones = jnp.full((L,), 1.0, dtype=jnp.float32)

    def start_in(g):
        return pltpu.async_copy(
            state_hbm.at[pl.ds(base + g * CHROWS, CHROWS)],
            bufs[g % NBUF], sin[g % NBUF])

    h_in = {}
    h_out = {}
    for g in range(IN_AHEAD):
        h_in[g] = start_in(g)
    for g in range(NCH):
        if g >= OUT_WAIT:
            h_out[g - OUT_WAIT].wait()
        nxt = g + IN_AHEAD
        if IN_AHEAD <= nxt < NCH:
            h_in[nxt] = start_in(nxt)
        h_in[g].wait()
        cvec = choice_v[pl.ds(g * CHROWS, L)] & (M - 1)
        b = g % NBUF
        plsc.addupdate_scatter(bufs[b], [row_idx, cvec], ones, mask=mask)
        h_out[g] = pltpu.async_copy(
            bufs[b], out_hbm.at[pl.ds(base + g * CHROWS, CHROWS)], sout[b])
    for g in range(NCH - OUT_WAIT, NCH):
        h_out[g].wait()


_sc_kernel = functools.partial(
    pl.kernel,
    out_type=jax.ShapeDtypeStruct((B, M), jnp.float32),
    mesh=plsc.VectorSubcoreMesh(
        core_axis_name="c", subcore_axis_name="s", num_cores=2, num_subcores=16
    ),
    compiler_params=pltpu.CompilerParams(needs_layout_passes=False),
    scratch_types=(
        [pltpu.VMEM((RPW + L, ), jnp.int32)]
        + [pltpu.VMEM((CHROWS, M), jnp.float32)] * NBUF
        + [pltpu.SemaphoreType.DMA] * (2 * NBUF)
    ),
)(_sc_body)


def kernel(state, choice):
    return _sc_kernel(state, choice.astype(jnp.int32))


# speedup vs baseline: 1.0150x; 1.0150x over previous
import functools

import jax
import jax.numpy as jnp
from jax import lax
from jax.experimental import pallas as pl
from jax.experimental.pallas import tpu as pltpu
from jax.experimental.pallas import tpu_sc as plsc

B = 16384
M = 4096
NW = 32
RPW = B // NW
L = 16
CHROWS = 8
NCH = RPW // CHROWS
NBUF = 3
IN_AHEAD = 1
OUT_WAIT = NBUF - IN_AHEAD


def _sc_body(state_hbm, choice_hbm, out_hbm, choice_v, *rest):
    bufs = list(rest[:NBUF])
    sin = list(rest[NBUF:2 * NBUF])
    sout = list(rest[2 * NBUF:3 * NBUF])
    wid = lax.axis_index("s") * 2 + lax.axis_index("c")
    base = wid * RPW

    lane = lax.iota(jnp.int32, L)
    row_idx = lane & (CHROWS - 1)
    mask = lane < CHROWS
    ones = jnp.full((L,), 1.0, dtype=jnp.float32)

    def start_in(g):
        return pltpu.async_copy(
            state_hbm.at[pl.ds(base + g * CHROWS, CHROWS)],
            bufs[g % NBUF], sin[g % NBUF])

    h_in = {}
    h_out = {}
    for g in range(IN_AHEAD):
        h_in[g] = start_in(g)

    pltpu.sync_copy(choice_hbm.at[pl.ds(base, RPW)], choice_v.at[pl.ds(0, RPW)])
    for g in range(NCH):
        if g >= OUT_WAIT:
            h_out[g - OUT_WAIT].wait()
        nxt = g + IN_AHEAD
        if IN_AHEAD <= nxt < NCH:
            h_in[nxt] = start_in(nxt)
        h_in[g].wait()
        cvec = choice_v[pl.ds(g * CHROWS, L)] & (M - 1)
        b = g % NBUF
        plsc.addupdate_scatter(bufs[b], [row_idx, cvec], ones, mask=mask)
        h_out[g] = pltpu.async_copy(
            bufs[b], out_hbm.at[pl.ds(base + g * CHROWS, CHROWS)], sout[b])
    for g in range(NCH - OUT_WAIT, NCH):
        h_out[g].wait()


_sc_kernel = functools.partial(
    pl.kernel,
    out_type=jax.ShapeDtypeStruct((B, M), jnp.float32),
    mesh=plsc.VectorSubcoreMesh(
        core_axis_name="c", subcore_axis_name="s", num_cores=2, num_subcores=16
    ),
    compiler_params=pltpu.CompilerParams(needs_layout_passes=False),
    scratch_types=(
        [pltpu.VMEM((RPW + L, ), jnp.int32)]
        + [pltpu.VMEM((CHROWS, M), jnp.float32)] * NBUF
        + [pltpu.SemaphoreType.DMA] * (2 * NBUF)
    ),
)(_sc_body)


def kernel(state, choice):
    return _sc_kernel(state, choice.astype(jnp.int32))
